# Initial kernel scaffold; baseline (speedup 1.0000x reference)
import jax
import jax.numpy as jnp
from jax.experimental import pallas as pl


def kernel(h, r, t, ent_table, rel_table):
    raise NotImplementedError("write your pallas kernel here")


# speedup vs baseline: 2.4947x; 2.4947x over previous
import functools

import jax
import jax.numpy as jnp
from jax import lax
from jax.experimental import pallas as pl
from jax.experimental.pallas import tpu as pltpu
from jax.experimental.pallas import tpu_sc as plsc

_INFO = plsc.get_sparse_core_info()
_NC = _INFO.num_cores
_NS = _INFO.num_subcores
_NW = _NC * _NS

_BATCH = 16384
_HIDDEN = 128
_BPW = _BATCH // _NW
_CHUNK = 128
_NCHUNK = _BPW // _CHUNK


def _trans_e_body(h_idx_hbm, r_idx_hbm, t_idx_hbm, ent_hbm, rel_hbm,
                  h_out, t_out, r_out,
                  idx_v, rows_v, gsem):
    wid = lax.axis_index("s") * _NC + lax.axis_index("c")
    base = wid * _BPW

    def do_table(idx_hbm, table_hbm, out_hbm):
        pltpu.sync_copy(idx_hbm.at[pl.ds(wid * _NCHUNK, _NCHUNK)], idx_v)
        cps = [
            pltpu.async_copy(
                table_hbm.at[idx_v.at[j]],
                rows_v.at[pl.ds(j * _CHUNK, _CHUNK)],
                gsem,
            )
            for j in range(_NCHUNK)
        ]
        for cp in cps:
            cp.wait()
        pltpu.sync_copy(rows_v, out_hbm.at[pl.ds(base, _BPW)])

    do_table(h_idx_hbm, ent_hbm, h_out)
    do_table(t_idx_hbm, ent_hbm, t_out)
    do_table(r_idx_hbm, rel_hbm, r_out)


@jax.jit
def _trans_e(h2, r2, t2, ent_table, rel_table):
    out = jax.ShapeDtypeStruct((_BATCH, _HIDDEN), jnp.float32)
    return pl.kernel(
        _trans_e_body,
        out_type=(out, out, out),
        mesh=plsc.VectorSubcoreMesh(core_axis_name="c", subcore_axis_name="s"),
        scratch_types=[
            pltpu.VMEM((_NCHUNK, _CHUNK), jnp.int32),
            pltpu.VMEM((_BPW, _HIDDEN), jnp.float32),
            pltpu.SemaphoreType.DMA,
        ],
    )(h2, r2, t2, ent_table, rel_table)


def kernel(h, r, t, ent_table, rel_table):
    shape2d = (_BATCH // _CHUNK, _CHUNK)
    h2 = h.reshape(shape2d)
    r2 = r.reshape(shape2d)
    t2 = t.reshape(shape2d)
    return _trans_e(h2, r2, t2, ent_table, rel_table)


# speedup vs baseline: 2.5353x; 1.0163x over previous
import jax
import jax.numpy as jnp
from jax import lax
from jax.experimental import pallas as pl
from jax.experimental.pallas import tpu as pltpu
from jax.experimental.pallas import tpu_sc as plsc

_INFO = plsc.get_sparse_core_info()
_NC = _INFO.num_cores
_NS = _INFO.num_subcores
_NW = _NC * _NS

_BATCH = 16384
_HIDDEN = 128
_BPW = _BATCH // _NW
_CHUNK = 128
_NCHUNK = _BPW // _CHUNK
_UNITS = 3 * _NCHUNK
_NBUF = 6
_LAG = 5


def _trans_e_body(h_idx_hbm, r_idx_hbm, t_idx_hbm, ent_hbm, rel_hbm,
                  h_out, t_out, r_out,
                  idx_v, rows_v, *sems):
    gsems = sems[:_NBUF]
    wsems = sems[_NBUF:]
    wid = lax.axis_index("s") * _NC + lax.axis_index("c")
    base = wid * _BPW

    pltpu.sync_copy(h_idx_hbm.at[pl.ds(wid * _NCHUNK, _NCHUNK)],
                    idx_v.at[pl.ds(0, _NCHUNK)])
    pltpu.sync_copy(t_idx_hbm.at[pl.ds(wid * _NCHUNK, _NCHUNK)],
                    idx_v.at[pl.ds(_NCHUNK, _NCHUNK)])
    pltpu.sync_copy(r_idx_hbm.at[pl.ds(wid * _NCHUNK, _NCHUNK)],
                    idx_v.at[pl.ds(2 * _NCHUNK, _NCHUNK)])

    tables = [ent_hbm] * _NCHUNK + [ent_hbm] * _NCHUNK + [rel_hbm] * _NCHUNK
    outs = [h_out] * _NCHUNK + [t_out] * _NCHUNK + [r_out] * _NCHUNK

    gcp = [None] * _NBUF
    wcp = [None] * _NBUF
    for step in range(_UNITS + _LAG):
        u = step
        if u < _UNITS:
            b = u % _NBUF
            if wcp[b] is not None:
                wcp[b].wait()
            gcp[b] = pltpu.async_copy(
                tables[u].at[idx_v.at[u]], rows_v.at[b], gsems[b])
        v = step - _LAG
        if v >= 0:
            b = v % _NBUF
            gcp[b].wait()
            wcp[b] = pltpu.async_copy(
                rows_v.at[b],
                outs[v].at[pl.ds(base + (v % _NCHUNK) * _CHUNK, _CHUNK)],
                wsems[b])
    for b in range(_NBUF):
        if wcp[b] is not None:
            wcp[b].wait()


@jax.jit
def _trans_e(h2, r2, t2, ent_table, rel_table):
    out = jax.ShapeDtypeStruct((_BATCH, _HIDDEN), jnp.float32)
    return pl.kernel(
        _trans_e_body,
        out_type=(out, out, out),
        mesh=plsc.VectorSubcoreMesh(core_axis_name="c", subcore_axis_name="s"),
        scratch_types=(
            [pltpu.VMEM((_UNITS, _CHUNK), jnp.int32),
             pltpu.VMEM((_NBUF, _CHUNK, _HIDDEN), jnp.float32)]
            + [pltpu.SemaphoreType.DMA] * (2 * _NBUF)
        ),
    )(h2, r2, t2, ent_table, rel_table)


def kernel(h, r, t, ent_table, rel_table):
    shape2d = (_BATCH // _CHUNK, _CHUNK)
    h2 = h.reshape(shape2d)
    r2 = r.reshape(shape2d)
    t2 = t.reshape(shape2d)
    return _trans_e(h2, r2, t2, ent_table, rel_table)


# speedup vs baseline: 2.6499x; 1.0452x over previous
import jax
import jax.numpy as jnp
from jax import lax
from jax.experimental import pallas as pl
from jax.experimental.pallas import tpu as pltpu
from jax.experimental.pallas import tpu_sc as plsc

_INFO = plsc.get_sparse_core_info()
_NC = _INFO.num_cores
_NS = _INFO.num_subcores
_NW = _NC * _NS

_BATCH = 16384
_HIDDEN = 128
_BPW = _BATCH // _NW
_CHUNK = 128
_NCHUNK = _BPW // _CHUNK
_UNITS = 2 * _NCHUNK
_NBUF = 6
_LAG = 5

_RV = 1024
_RBLK = 2048


def _ht_body(h_idx_hbm, t_idx_hbm, ent_hbm,
             h_out, t_out,
             idx_v, rows_v, *sems):
    gsems = sems[:_NBUF]
    wsems = sems[_NBUF:]
    wid = lax.axis_index("s") * _NC + lax.axis_index("c")
    base = wid * _BPW

    pltpu.sync_copy(h_idx_hbm.at[pl.ds(wid * _NCHUNK, _NCHUNK)],
                    idx_v.at[pl.ds(0, _NCHUNK)])
    pltpu.sync_copy(t_idx_hbm.at[pl.ds(wid * _NCHUNK, _NCHUNK)],
                    idx_v.at[pl.ds(_NCHUNK, _NCHUNK)])

    outs = [h_out] * _NCHUNK + [t_out] * _NCHUNK

    gcp = [None] * _NBUF
    wcp = [None] * _NBUF
    for step in range(_UNITS + _LAG):
        u = step
        if u < _UNITS:
            b = u % _NBUF
            if wcp[b] is not None:
                wcp[b].wait()
            gcp[b] = pltpu.async_copy(
                ent_hbm.at[idx_v.at[u]], rows_v.at[b], gsems[b])
        v = step - _LAG
        if v >= 0:
            b = v % _NBUF
            gcp[b].wait()
            wcp[b] = pltpu.async_copy(
                rows_v.at[b],
                outs[v].at[pl.ds(base + (v % _NCHUNK) * _CHUNK, _CHUNK)],
                wsems[b])
    for b in range(_NBUF):
        if wcp[b] is not None:
            wcp[b].wait()


def _r_body(r_ref, rel_ref, out_ref):
    idx = r_ref[...]
    iota = lax.broadcasted_iota(jnp.int32, (_RBLK, _RV), 1)
    onehot = jnp.where(idx == iota, 1.0, 0.0)
    out_ref[...] = jnp.dot(onehot, rel_ref[...],
                           preferred_element_type=jnp.float32)


@jax.jit
def _trans_e(h2, t2, r_col, ent_table, rel_pad):
    out = jax.ShapeDtypeStruct((_BATCH, _HIDDEN), jnp.float32)
    h_emb, t_emb = pl.kernel(
        _ht_body,
        out_type=(out, out),
        mesh=plsc.VectorSubcoreMesh(core_axis_name="c", subcore_axis_name="s"),
        scratch_types=(
            [pltpu.VMEM((_UNITS, _CHUNK), jnp.int32),
             pltpu.VMEM((_NBUF, _CHUNK, _HIDDEN), jnp.float32)]
            + [pltpu.SemaphoreType.DMA] * (2 * _NBUF)
        ),
    )(h2, t2, ent_table)
    r_emb = pl.pallas_call(
        _r_body,
        grid=(_BATCH // _RBLK,),
        in_specs=[
            pl.BlockSpec((_RBLK, 1), lambda i: (i, 0)),
            pl.BlockSpec((_RV, _HIDDEN), lambda i: (0, 0)),
        ],
        out_specs=pl.BlockSpec((_RBLK, _HIDDEN), lambda i: (i, 0)),
        out_shape=out,
    )(r_col, rel_pad)
    return h_emb, t_emb, r_emb


def kernel(h, r, t, ent_table, rel_table):
    shape2d = (_BATCH // _CHUNK, _CHUNK)
    h2 = h.reshape(shape2d)
    t2 = t.reshape(shape2d)
    r_col = r.reshape(_BATCH, 1)
    rel_pad = jnp.pad(rel_table, ((0, _RV - rel_table.shape[0]), (0, 0)))
    return _trans_e(h2, t2, r_col, ent_table, rel_pad)
